# baseline (speedup 1.0000x reference)
import functools

import jax
import jax.numpy as jnp
from jax import lax
from jax.experimental import pallas as pl
from jax.experimental.pallas import tpu as pltpu
from jax.experimental.pallas import tpu_sc as plsc

BATCH = 16384
D = 64
NV = BATCH // 16

_NC, _NS = 2, 16
_NW = _NC * _NS
NBLK = 7813
RPW = 245
NSLOT = 512
NSCAT = NSLOT * D // 128
TRASH = BATCH
OUT1D = (BATCH + 128) * D
NRING = 8


def _s0(v):
    return lax.squeeze(lax.slice(v, (0,), (1,)), (0,))


def _iota16():
    return lax.broadcasted_iota(jnp.int32, (16,), 0)


def _gather_one_table(tt_hbm, idx_hbm, out_hbm, idxall, cu, cp, glu, glp,
                      fbuf, bufs, sems, ssem, hist_s, cum_s,
                      occb_s, occs_s, wid, lo, hi):
    iota = _iota16()
    sent_b = hi

    pltpu.sync_copy(idx_hbm, idxall)

    def round_body(carry):
        vres, r = carry

        def p1_cond(c):
            v, off = c
            return (v < NV) & (off < NSLOT - 15)

        def p1_body(c):
            v, off = c
            u = idxall[pl.ds(v * 16, 16)]
            p = iota + v * 16
            b = u >> 7
            m = (b >= lo) & (b < hi)
            plsc.store_compressed(cu.at[pl.ds(off, 16)], u, mask=m)
            plsc.store_compressed(cp.at[pl.ds(off, 16)], p, mask=m)
            cnt = _s0(plsc.all_reduce_population_count(m))
            return v + 1, off + cnt

        vres2, n_t = lax.while_loop(p1_cond, p1_body, (vres, 0))

        def p2_zero(i, _):
            hist_s[i] = 0
            return 0
        lax.fori_loop(0, RPW, p2_zero, 0)

        def p2_body(j, _):
            u = _s0(cu[pl.ds(j, 16)])
            bb = (u >> 7) - lo
            hist_s[bb] = hist_s[bb] + 1
            return 0
        lax.fori_loop(0, n_t, p2_body, 0)

        occb_s[0] = lo
        occs_s[0] = 0

        def p3_body(i, c):
            run, no = c
            h = hist_s[i]
            cum_s[i] = run

            @pl.when(h > 0)
            def _():
                occb_s[no] = i + lo
                occs_s[no] = run

            return run + h, jnp.where(h > 0, no + 1, no)

        _, nocc = lax.fori_loop(0, hi - lo, p3_body, (0, 0))

        for w in range(NSLOT // 16):
            glp[pl.ds(16 * w, 16)] = jnp.full((16,), TRASH, jnp.int32) + (
                iota + 16 * w) % 128

        def p4_body(j, c):
            u = _s0(cu[pl.ds(j, 16)])
            p = _s0(cp[pl.ds(j, 16)])
            bb = (u >> 7) - lo
            slot = cum_s[bb]
            cum_s[bb] = slot + 1
            m0 = iota == 0
            sv = jnp.full((16,), 0, jnp.int32) + slot
            plsc.store_scatter(glu, [sv], jnp.full((16,), 0, jnp.int32) + u,
                               mask=m0)
            plsc.store_scatter(glp, [sv], jnp.full((16,), 0, jnp.int32) + p,
                               mask=m0)
            return c
        lax.fori_loop(0, n_t, p4_body, 0)

        def fetch(k, buf, sem):
            kk = jnp.maximum(jnp.minimum(k, nocc - 1), 0)
            off = pl.multiple_of(occb_s[kk] * 128, 128)
            pltpu.async_copy(tt_hbm.at[pl.ds(0, 32), pl.ds(off, 128)],
                             buf.at[pl.ds(0, 32)], sem)
            pltpu.async_copy(tt_hbm.at[pl.ds(32, 32), pl.ds(off, 128)],
                             buf.at[pl.ds(32, 32)], sem)

        for b in range(NRING):
            fetch(b, bufs[b], sems[b])

        def chunk_body(cidx, c):
            for b in range(NRING):
                k = cidx * NRING + b
                for hh in range(2):
                    pltpu.make_async_copy(
                        tt_hbm.at[pl.ds(32 * hh, 32), pl.ds(0, 128)],
                        bufs[b].at[pl.ds(32 * hh, 32)], sems[b]).wait()

                @pl.when(k < nocc)
                def _(k=k, b=b):
                    kk = jnp.maximum(jnp.minimum(k, nocc - 1), 0)
                    st = occs_s[kk]
                    cnt = hist_s[occb_s[kk] - lo]

                    def ex_body(j, cc2):
                        u = _s0(glu[pl.ds(j, 16)])
                        ln = u & 127
                        lnv = jnp.full((16,), 0, jnp.int32) + ln
                        for cc in range(4):
                            vals = plsc.load_gather(
                                bufs[b], [iota + cc * 16, lnv])
                            fbuf[pl.ds(j * D + cc * 16, 16)] = vals
                        return cc2
                    lax.fori_loop(st, st + cnt, ex_body, 0)

                fetch(k + NRING, bufs[b], sems[b])
            return c
        lax.fori_loop(0, (nocc + NRING - 1) // NRING, chunk_body, 0)
        for b in range(NRING):
            for hh in range(2):
                pltpu.make_async_copy(
                    tt_hbm.at[pl.ds(32 * hh, 32), pl.ds(0, 128)],
                    bufs[b].at[pl.ds(32 * hh, 32)], sems[b]).wait()

        def p7_fire(j, carry):
            p = _s0(glp[pl.ds(j, 16)])
            src = pl.multiple_of(j * D, D)
            dst = pl.multiple_of(p * D, D)
            pltpu.async_copy(
                fbuf.at[pl.ds(src, D)], out_hbm.at[pl.ds(dst, D)], ssem)
            return carry
        lax.fori_loop(0, n_t, p7_fire, 0)

        def p7_drain(j, carry):
            pltpu.make_async_copy(
                fbuf.at[pl.ds(0, D)], out_hbm.at[pl.ds(0, D)], ssem).wait()
            return carry
        lax.fori_loop(0, n_t, p7_drain, 0)

        return vres2, r + 1

    def round_cond(carry):
        vres, r = carry
        return vres < NV

    lax.while_loop(round_cond, round_body, (0, 0))


@functools.cache
def _make_sc_gather():
    mesh = plsc.VectorSubcoreMesh(core_axis_name="c", subcore_axis_name="s")

    @functools.partial(
        pl.kernel,
        out_type=(
            jax.ShapeDtypeStruct((OUT1D,), jnp.float32),
            jax.ShapeDtypeStruct((OUT1D,), jnp.float32),
        ),
        mesh=mesh,
        compiler_params=pltpu.CompilerParams(needs_layout_passes=False),
        scratch_types=[
            pltpu.VMEM((BATCH,), jnp.int32),
            pltpu.VMEM((NSLOT + 16,), jnp.int32),
            pltpu.VMEM((NSLOT + 16,), jnp.int32),
            pltpu.VMEM((NSLOT + 16,), jnp.int32),
            pltpu.VMEM((NSLOT + 16,), jnp.int32),
            pltpu.VMEM((NSLOT * D,), jnp.float32),
            pltpu.VMEM((D, 128), jnp.float32),
            pltpu.VMEM((D, 128), jnp.float32),
            pltpu.VMEM((D, 128), jnp.float32),
            pltpu.VMEM((D, 128), jnp.float32),
            pltpu.VMEM((D, 128), jnp.float32),
            pltpu.VMEM((D, 128), jnp.float32),
            pltpu.VMEM((D, 128), jnp.float32),
            pltpu.VMEM((D, 128), jnp.float32),
            pltpu.SemaphoreType.DMA,
            pltpu.SemaphoreType.DMA,
            pltpu.SemaphoreType.DMA,
            pltpu.SemaphoreType.DMA,
            pltpu.SemaphoreType.DMA,
            pltpu.SemaphoreType.DMA,
            pltpu.SemaphoreType.DMA,
            pltpu.SemaphoreType.DMA,
            pltpu.SemaphoreType.DMA,

            pltpu.SMEM((256,), jnp.int32),
            pltpu.SMEM((256,), jnp.int32),
            pltpu.SMEM((256,), jnp.int32),
            pltpu.SMEM((256,), jnp.int32),
        ],
    )
    def sc_gather(tt_u, tt_b, uidx_hbm, bidx_hbm, ou_hbm, ob_hbm,
                  idxall, cu, cp, glu, glp, fbuf,
                  buf0, buf1, buf2, buf3, buf4, buf5, buf6, buf7,
                  sem0, sem1, sem2, sem3, sem4, sem5, sem6, sem7, ssem,
                  hist_s, cum_s, occb_s, occs_s):
        wid = lax.axis_index("s") * _NC + lax.axis_index("c")
        lo = wid * RPW
        hi = jnp.minimum(lo + RPW, NBLK)
        bufs = (buf0, buf1, buf2, buf3, buf4, buf5, buf6, buf7)
        sems = (sem0, sem1, sem2, sem3, sem4, sem5, sem6, sem7)
        for tt, idx, out in ((tt_u, uidx_hbm, ou_hbm),
                             (tt_b, bidx_hbm, ob_hbm)):
            _gather_one_table(tt, idx, out, idxall, cu, cp, glu, glp,
                              fbuf, bufs, sems, ssem,
                              hist_s, cum_s, occb_s, occs_s, wid, lo, hi)

    return sc_gather


_BM = 1024


def _dot(a, b):
    return jax.lax.dot(a, b, preferred_element_type=jnp.float32)


def _mlp_body(ue_ref, be_ref, w1a_ref, w1b_ref, b1_ref, w2_ref, b2_ref,
              w3_ref, b3_ref, w4_ref, b4_ref, out_ref):
    ue = ue_ref[...].astype(jnp.bfloat16)
    be = be_ref[...].astype(jnp.bfloat16)
    h = _dot(ue, w1a_ref[...]) + _dot(be, w1b_ref[...]) + b1_ref[...]
    h = jnp.maximum(h, 0.0).astype(jnp.bfloat16)
    h = jnp.maximum(_dot(h, w2_ref[...]) + b2_ref[...], 0.0).astype(jnp.bfloat16)
    h = jnp.maximum(_dot(h, w3_ref[...]) + b3_ref[...], 0.0)
    out_ref[...] = jnp.sum(h * w4_ref[...], axis=1) + b4_ref[0]


def _mlp(ue, be, W1a, W1b, b1, W2, b2, W3, b3, w4row, b4):
    grid = (BATCH // _BM,)
    full = lambda i: (0, 0)
    return pl.pallas_call(
        _mlp_body,
        grid=grid,
        in_specs=[
            pl.BlockSpec((_BM, D), lambda i: (i, 0)),
            pl.BlockSpec((_BM, D), lambda i: (i, 0)),
            pl.BlockSpec((D, 1024), full),
            pl.BlockSpec((D, 1024), full),
            pl.BlockSpec((1, 1024), full),
            pl.BlockSpec((1024, 512), full),
            pl.BlockSpec((1, 512), full),
            pl.BlockSpec((512, 256), full),
            pl.BlockSpec((1, 256), full),
            pl.BlockSpec((1, 256), full),
            pl.BlockSpec(memory_space=pltpu.SMEM),
        ],
        out_specs=pl.BlockSpec((_BM,), lambda i: (i,)),
        out_shape=jax.ShapeDtypeStruct((BATCH,), jnp.float32),
    )(ue, be, W1a, W1b, b1, W2, b2, W3, b3, w4row, b4)


def kernel(users, businesses, user_table, business_table,
           W1, b1, W2, b2, W3, b3, W4, b4):
    uidx = users.astype(jnp.int32)
    bidx = businesses.astype(jnp.int32)
    ou, ob = _make_sc_gather()(user_table.T, business_table.T, uidx, bidx)
    ue = ou.reshape(BATCH + 128, D)[:BATCH]
    be = ob.reshape(BATCH + 128, D)[:BATCH]
    W1a = W1[:D].astype(jnp.bfloat16)
    W1b = W1[D:].astype(jnp.bfloat16)
    w4row = W4.reshape(1, 256)
    return _mlp(ue, be, W1a, W1b, b1.reshape(1, 1024),
                W2.astype(jnp.bfloat16), b2.reshape(1, 512),
                W3.astype(jnp.bfloat16), b3.reshape(1, 256), w4row, b4)


# speedup vs baseline: 1.0968x; 1.0968x over previous
import functools

import jax
import jax.numpy as jnp
from jax import lax
from jax.experimental import pallas as pl
from jax.experimental.pallas import tpu as pltpu
from jax.experimental.pallas import tpu_sc as plsc

BATCH = 16384
D = 64
NV = BATCH // 16

_NC, _NS = 2, 16
_NW = _NC * _NS
NBLK = 7813
RPW = 245
NSLOT = 768
NSCAT = NSLOT * D // 128
TRASH = BATCH
OUT1D = (BATCH + 128) * D
NRING = 7


def _s0(v):
    return lax.squeeze(lax.slice(v, (0,), (1,)), (0,))


def _iota16():
    return lax.broadcasted_iota(jnp.int32, (16,), 0)


def _gather_one_table(tt_hbm, idx_hbm, out_hbm, idxall, cu, cp, glu, glp,
                      fbuf, bufs, sems, ssem, hist_s, cum_s,
                      occb_s, occs_s, wid, lo, hi):
    iota = _iota16()
    sent_b = hi

    pltpu.sync_copy(idx_hbm, idxall)

    def round_body(carry):
        vres, r = carry

        def p1_cond(c):
            v, off = c
            return (v < NV) & (off < NSLOT - 15)

        def p1_body(c):
            v, off = c
            u = idxall[pl.ds(v * 16, 16)]
            p = iota + v * 16
            b = u >> 7
            m = (b >= lo) & (b < hi)
            plsc.store_compressed(cu.at[pl.ds(off, 16)], u, mask=m)
            plsc.store_compressed(cp.at[pl.ds(off, 16)], p, mask=m)
            cnt = _s0(plsc.all_reduce_population_count(m))
            return v + 1, off + cnt

        vres2, n_t = lax.while_loop(p1_cond, p1_body, (vres, 0))

        def p2_zero(i, _):
            hist_s[i] = 0
            return 0
        lax.fori_loop(0, RPW, p2_zero, 0)

        def p2_body(j, _):
            u = _s0(cu[pl.ds(j, 16)])
            bb = (u >> 7) - lo
            hist_s[bb] = hist_s[bb] + 1
            return 0
        lax.fori_loop(0, n_t, p2_body, 0)

        occb_s[0] = lo
        occs_s[0] = 0

        def p3_body(i, c):
            run, no = c
            h = hist_s[i]
            cum_s[i] = run

            @pl.when(h > 0)
            def _():
                occb_s[no] = i + lo
                occs_s[no] = run

            return run + h, jnp.where(h > 0, no + 1, no)

        _, nocc = lax.fori_loop(0, hi - lo, p3_body, (0, 0))

        for w in range(NSLOT // 16):
            glp[pl.ds(16 * w, 16)] = jnp.full((16,), TRASH, jnp.int32) + (
                iota + 16 * w) % 128

        def p4_body(j, c):
            u = _s0(cu[pl.ds(j, 16)])
            p = _s0(cp[pl.ds(j, 16)])
            bb = (u >> 7) - lo
            slot = cum_s[bb]
            cum_s[bb] = slot + 1
            m0 = iota == 0
            sv = jnp.full((16,), 0, jnp.int32) + slot
            plsc.store_scatter(glu, [sv], jnp.full((16,), 0, jnp.int32) + u,
                               mask=m0)
            plsc.store_scatter(glp, [sv], jnp.full((16,), 0, jnp.int32) + p,
                               mask=m0)
            return c
        lax.fori_loop(0, n_t, p4_body, 0)

        def fetch(k, buf, sem):
            kk = jnp.maximum(jnp.minimum(k, nocc - 1), 0)
            off = pl.multiple_of(occb_s[kk] * 128, 128)
            pltpu.async_copy(tt_hbm.at[pl.ds(0, 32), pl.ds(off, 128)],
                             buf.at[pl.ds(0, 32)], sem)
            pltpu.async_copy(tt_hbm.at[pl.ds(32, 32), pl.ds(off, 128)],
                             buf.at[pl.ds(32, 32)], sem)

        for b in range(NRING):
            fetch(b, bufs[b], sems[b])

        def chunk_body(cidx, c):
            for b in range(NRING):
                k = cidx * NRING + b
                for hh in range(2):
                    pltpu.make_async_copy(
                        tt_hbm.at[pl.ds(32 * hh, 32), pl.ds(0, 128)],
                        bufs[b].at[pl.ds(32 * hh, 32)], sems[b]).wait()

                @pl.when(k < nocc)
                def _(k=k, b=b):
                    kk = jnp.maximum(jnp.minimum(k, nocc - 1), 0)
                    st = occs_s[kk]
                    cnt = hist_s[occb_s[kk] - lo]

                    def ex_body(j, cc2):
                        u = _s0(glu[pl.ds(j, 16)])
                        ln = u & 127
                        lnv = jnp.full((16,), 0, jnp.int32) + ln
                        for cc in range(4):
                            vals = plsc.load_gather(
                                bufs[b], [iota + cc * 16, lnv])
                            fbuf[pl.ds(j * D + cc * 16, 16)] = vals
                        return cc2
                    lax.fori_loop(st, st + cnt, ex_body, 0)

                fetch(k + NRING, bufs[b], sems[b])
            return c
        lax.fori_loop(0, (nocc + NRING - 1) // NRING, chunk_body, 0)
        for b in range(NRING):
            for hh in range(2):
                pltpu.make_async_copy(
                    tt_hbm.at[pl.ds(32 * hh, 32), pl.ds(0, 128)],
                    bufs[b].at[pl.ds(32 * hh, 32)], sems[b]).wait()

        def p7_fire(j, carry):
            p = _s0(glp[pl.ds(j, 16)])
            src = pl.multiple_of(j * D, D)
            dst = pl.multiple_of(p * D, D)
            pltpu.async_copy(
                fbuf.at[pl.ds(src, D)], out_hbm.at[pl.ds(dst, D)], ssem)
            return carry
        lax.fori_loop(0, n_t, p7_fire, 0)

        def p7_drain(j, carry):
            pltpu.make_async_copy(
                fbuf.at[pl.ds(0, D)], out_hbm.at[pl.ds(0, D)], ssem).wait()
            return carry
        lax.fori_loop(0, n_t, p7_drain, 0)

        return vres2, r + 1

    def round_cond(carry):
        vres, r = carry
        return vres < NV

    lax.while_loop(round_cond, round_body, (0, 0))


@functools.cache
def _make_sc_gather():
    mesh = plsc.VectorSubcoreMesh(core_axis_name="c", subcore_axis_name="s")

    @functools.partial(
        pl.kernel,
        out_type=(
            jax.ShapeDtypeStruct((OUT1D,), jnp.float32),
            jax.ShapeDtypeStruct((OUT1D,), jnp.float32),
        ),
        mesh=mesh,
        compiler_params=pltpu.CompilerParams(needs_layout_passes=False),
        scratch_types=[
            pltpu.VMEM((BATCH,), jnp.int32),
            pltpu.VMEM((NSLOT + 16,), jnp.int32),
            pltpu.VMEM((NSLOT + 16,), jnp.int32),
            pltpu.VMEM((NSLOT + 16,), jnp.int32),
            pltpu.VMEM((NSLOT + 16,), jnp.int32),
            pltpu.VMEM((NSLOT * D,), jnp.float32),
            pltpu.VMEM((D, 128), jnp.float32),
            pltpu.VMEM((D, 128), jnp.float32),
            pltpu.VMEM((D, 128), jnp.float32),
            pltpu.VMEM((D, 128), jnp.float32),
            pltpu.VMEM((D, 128), jnp.float32),
            pltpu.VMEM((D, 128), jnp.float32),
            pltpu.VMEM((D, 128), jnp.float32),
            pltpu.SemaphoreType.DMA,
            pltpu.SemaphoreType.DMA,
            pltpu.SemaphoreType.DMA,
            pltpu.SemaphoreType.DMA,
            pltpu.SemaphoreType.DMA,
            pltpu.SemaphoreType.DMA,
            pltpu.SemaphoreType.DMA,
            pltpu.SemaphoreType.DMA,

            pltpu.SMEM((256,), jnp.int32),
            pltpu.SMEM((256,), jnp.int32),
            pltpu.SMEM((256,), jnp.int32),
            pltpu.SMEM((256,), jnp.int32),
        ],
    )
    def sc_gather(tt_u, tt_b, uidx_hbm, bidx_hbm, ou_hbm, ob_hbm,
                  idxall, cu, cp, glu, glp, fbuf,
                  buf0, buf1, buf2, buf3, buf4, buf5, buf6,
                  sem0, sem1, sem2, sem3, sem4, sem5, sem6, ssem,
                  hist_s, cum_s, occb_s, occs_s):
        wid = lax.axis_index("s") * _NC + lax.axis_index("c")
        lo = wid * RPW
        hi = jnp.minimum(lo + RPW, NBLK)
        bufs = (buf0, buf1, buf2, buf3, buf4, buf5, buf6)
        sems = (sem0, sem1, sem2, sem3, sem4, sem5, sem6)
        for tt, idx, out in ((tt_u, uidx_hbm, ou_hbm),
                             (tt_b, bidx_hbm, ob_hbm)):
            _gather_one_table(tt, idx, out, idxall, cu, cp, glu, glp,
                              fbuf, bufs, sems, ssem,
                              hist_s, cum_s, occb_s, occs_s, wid, lo, hi)

    return sc_gather


_BM = 1024


def _dot(a, b):
    return jax.lax.dot(a, b, preferred_element_type=jnp.float32)


def _mlp_body(ue_ref, be_ref, w1a_ref, w1b_ref, b1_ref, w2_ref, b2_ref,
              w3_ref, b3_ref, w4_ref, b4_ref, out_ref):
    ue = ue_ref[...].astype(jnp.bfloat16)
    be = be_ref[...].astype(jnp.bfloat16)
    h = _dot(ue, w1a_ref[...]) + _dot(be, w1b_ref[...]) + b1_ref[...]
    h = jnp.maximum(h, 0.0).astype(jnp.bfloat16)
    h = jnp.maximum(_dot(h, w2_ref[...]) + b2_ref[...], 0.0).astype(jnp.bfloat16)
    h = jnp.maximum(_dot(h, w3_ref[...]) + b3_ref[...], 0.0)
    out_ref[...] = jnp.sum(h * w4_ref[...], axis=1) + b4_ref[0]


def _mlp(ue, be, W1a, W1b, b1, W2, b2, W3, b3, w4row, b4):
    grid = (BATCH // _BM,)
    full = lambda i: (0, 0)
    return pl.pallas_call(
        _mlp_body,
        grid=grid,
        in_specs=[
            pl.BlockSpec((_BM, D), lambda i: (i, 0)),
            pl.BlockSpec((_BM, D), lambda i: (i, 0)),
            pl.BlockSpec((D, 1024), full),
            pl.BlockSpec((D, 1024), full),
            pl.BlockSpec((1, 1024), full),
            pl.BlockSpec((1024, 512), full),
            pl.BlockSpec((1, 512), full),
            pl.BlockSpec((512, 256), full),
            pl.BlockSpec((1, 256), full),
            pl.BlockSpec((1, 256), full),
            pl.BlockSpec(memory_space=pltpu.SMEM),
        ],
        out_specs=pl.BlockSpec((_BM,), lambda i: (i,)),
        out_shape=jax.ShapeDtypeStruct((BATCH,), jnp.float32),
    )(ue, be, W1a, W1b, b1, W2, b2, W3, b3, w4row, b4)


def kernel(users, businesses, user_table, business_table,
           W1, b1, W2, b2, W3, b3, W4, b4):
    uidx = users.astype(jnp.int32)
    bidx = businesses.astype(jnp.int32)
    ou, ob = _make_sc_gather()(user_table.T, business_table.T, uidx, bidx)
    ue = ou.reshape(BATCH + 128, D)[:BATCH]
    be = ob.reshape(BATCH + 128, D)[:BATCH]
    W1a = W1[:D].astype(jnp.bfloat16)
    W1b = W1[D:].astype(jnp.bfloat16)
    w4row = W4.reshape(1, 256)
    return _mlp(ue, be, W1a, W1b, b1.reshape(1, 1024),
                W2.astype(jnp.bfloat16), b2.reshape(1, 512),
                W3.astype(jnp.bfloat16), b3.reshape(1, 256), w4row, b4)


# speedup vs baseline: 1.1518x; 1.0501x over previous
import functools

import jax
import jax.numpy as jnp
from jax import lax
from jax.experimental import pallas as pl
from jax.experimental.pallas import tpu as pltpu
from jax.experimental.pallas import tpu_sc as plsc

BATCH = 16384
D = 64
NV = BATCH // 16

_NC, _NS = 2, 16
_NW = _NC * _NS
NBLK = 7813
RPW = 245
NSLOT = 768
NSCAT = NSLOT * D // 128
TRASH = BATCH
OUT1D = (BATCH + 128) * D
NRING = 7


def _s0(v):
    return lax.squeeze(lax.slice(v, (0,), (1,)), (0,))


def _iota16():
    return lax.broadcasted_iota(jnp.int32, (16,), 0)


def _gather_one_table(tt_hbm, idx_hbm, out_hbm, idxall, cu, cp, glu, glp,
                      fbuf, bufs, sems, ssem, hist_s, cum_s,
                      occb_s, occs_s, wid, lo, hi):
    iota = _iota16()
    sent_b = hi

    pltpu.sync_copy(idx_hbm, idxall)

    def round_body(carry):
        vres, r = carry

        def p1_cond(c):
            v, off = c
            return (v < NV) & (off < NSLOT - 15)

        def p1_body(c):
            v, off = c
            u = idxall[pl.ds(v * 16, 16)]
            p = iota + v * 16
            b = u >> 7
            m = (b >= lo) & (b < hi)
            plsc.store_compressed(cu.at[pl.ds(off, 16)], u, mask=m)
            plsc.store_compressed(cp.at[pl.ds(off, 16)], p, mask=m)
            cnt = _s0(plsc.all_reduce_population_count(m))
            return v + 1, off + cnt

        vres2, n_t = lax.while_loop(p1_cond, p1_body, (vres, 0))

        def p2_zero(i, _):
            hist_s[i] = 0
            return 0
        lax.fori_loop(0, RPW, p2_zero, 0)

        def p2_body(j, _):
            u = _s0(cu[pl.ds(j, 16)])
            bb = (u >> 7) - lo
            hist_s[bb] = hist_s[bb] + 1
            return 0
        lax.fori_loop(0, n_t, p2_body, 0)

        occb_s[0] = lo
        occs_s[0] = 0

        def p3_body(i, c):
            run, no = c
            h = hist_s[i]
            cum_s[i] = run

            @pl.when(h > 0)
            def _():
                occb_s[no] = i + lo
                occs_s[no] = run

            return run + h, jnp.where(h > 0, no + 1, no)

        _, nocc = lax.fori_loop(0, hi - lo, p3_body, (0, 0))

        for w in range(NSLOT // 16):
            glp[pl.ds(16 * w, 16)] = jnp.full((16,), TRASH, jnp.int32) + (
                iota + 16 * w) % 128

        def p4_body(j, c):
            u = _s0(cu[pl.ds(j, 16)])
            p = _s0(cp[pl.ds(j, 16)])
            bb = (u >> 7) - lo
            slot = cum_s[bb]
            cum_s[bb] = slot + 1
            m0 = iota == 0
            sv = jnp.full((16,), 0, jnp.int32) + slot
            plsc.store_scatter(glu, [sv], jnp.full((16,), 0, jnp.int32) + u,
                               mask=m0)
            plsc.store_scatter(glp, [sv], jnp.full((16,), 0, jnp.int32) + p,
                               mask=m0)
            return c
        lax.fori_loop(0, n_t, p4_body, 0)

        def fetch(k, buf, sem):
            kk = jnp.maximum(jnp.minimum(k, nocc - 1), 0)
            off = pl.multiple_of(occb_s[kk] * 128, 128)
            pltpu.async_copy(tt_hbm.at[pl.ds(0, 32), pl.ds(off, 128)],
                             buf.at[pl.ds(0, 32)], sem)
            pltpu.async_copy(tt_hbm.at[pl.ds(32, 32), pl.ds(off, 128)],
                             buf.at[pl.ds(32, 32)], sem)

        for b in range(NRING):
            fetch(b, bufs[b], sems[b])

        def chunk_body(cidx, c):
            for b in range(NRING):
                k = cidx * NRING + b
                for hh in range(2):
                    pltpu.make_async_copy(
                        tt_hbm.at[pl.ds(32 * hh, 32), pl.ds(0, 128)],
                        bufs[b].at[pl.ds(32 * hh, 32)], sems[b]).wait()

                @pl.when(k < nocc)
                def _(k=k, b=b):
                    kk = jnp.maximum(jnp.minimum(k, nocc - 1), 0)
                    st = occs_s[kk]
                    cnt = hist_s[occb_s[kk] - lo]

                    def ex_body(j, cc2):
                        u = _s0(glu[pl.ds(j, 16)])
                        ln = u & 127
                        lnv = jnp.full((16,), 0, jnp.int32) + ln
                        for cc in range(4):
                            vals = plsc.load_gather(
                                bufs[b], [iota + cc * 16, lnv])
                            fbuf[pl.ds(j * D + cc * 16, 16)] = vals
                        p = _s0(glp[pl.ds(j, 16)])
                        src = pl.multiple_of(j * D, D)
                        dst = pl.multiple_of(p * D, D)
                        pltpu.async_copy(fbuf.at[pl.ds(src, D)],
                                         out_hbm.at[pl.ds(dst, D)], ssem)
                        return cc2
                    lax.fori_loop(st, st + cnt, ex_body, 0)

                fetch(k + NRING, bufs[b], sems[b])
            return c
        lax.fori_loop(0, (nocc + NRING - 1) // NRING, chunk_body, 0)
        for b in range(NRING):
            for hh in range(2):
                pltpu.make_async_copy(
                    tt_hbm.at[pl.ds(32 * hh, 32), pl.ds(0, 128)],
                    bufs[b].at[pl.ds(32 * hh, 32)], sems[b]).wait()

        def p7_drain(j, carry):
            pltpu.make_async_copy(
                fbuf.at[pl.ds(0, D)], out_hbm.at[pl.ds(0, D)], ssem).wait()
            return carry
        lax.fori_loop(0, n_t, p7_drain, 0)

        return vres2, r + 1

    def round_cond(carry):
        vres, r = carry
        return vres < NV

    lax.while_loop(round_cond, round_body, (0, 0))


@functools.cache
def _make_sc_gather():
    mesh = plsc.VectorSubcoreMesh(core_axis_name="c", subcore_axis_name="s")

    @functools.partial(
        pl.kernel,
        out_type=(
            jax.ShapeDtypeStruct((OUT1D,), jnp.float32),
            jax.ShapeDtypeStruct((OUT1D,), jnp.float32),
        ),
        mesh=mesh,
        compiler_params=pltpu.CompilerParams(needs_layout_passes=False),
        scratch_types=[
            pltpu.VMEM((BATCH,), jnp.int32),
            pltpu.VMEM((NSLOT + 16,), jnp.int32),
            pltpu.VMEM((NSLOT + 16,), jnp.int32),
            pltpu.VMEM((NSLOT + 16,), jnp.int32),
            pltpu.VMEM((NSLOT + 16,), jnp.int32),
            pltpu.VMEM((NSLOT * D,), jnp.float32),
            pltpu.VMEM((D, 128), jnp.float32),
            pltpu.VMEM((D, 128), jnp.float32),
            pltpu.VMEM((D, 128), jnp.float32),
            pltpu.VMEM((D, 128), jnp.float32),
            pltpu.VMEM((D, 128), jnp.float32),
            pltpu.VMEM((D, 128), jnp.float32),
            pltpu.VMEM((D, 128), jnp.float32),
            pltpu.SemaphoreType.DMA,
            pltpu.SemaphoreType.DMA,
            pltpu.SemaphoreType.DMA,
            pltpu.SemaphoreType.DMA,
            pltpu.SemaphoreType.DMA,
            pltpu.SemaphoreType.DMA,
            pltpu.SemaphoreType.DMA,
            pltpu.SemaphoreType.DMA,

            pltpu.SMEM((256,), jnp.int32),
            pltpu.SMEM((256,), jnp.int32),
            pltpu.SMEM((256,), jnp.int32),
            pltpu.SMEM((256,), jnp.int32),
        ],
    )
    def sc_gather(tt_u, tt_b, uidx_hbm, bidx_hbm, ou_hbm, ob_hbm,
                  idxall, cu, cp, glu, glp, fbuf,
                  buf0, buf1, buf2, buf3, buf4, buf5, buf6,
                  sem0, sem1, sem2, sem3, sem4, sem5, sem6, ssem,
                  hist_s, cum_s, occb_s, occs_s):
        wid = lax.axis_index("s") * _NC + lax.axis_index("c")
        lo = wid * RPW
        hi = jnp.minimum(lo + RPW, NBLK)
        bufs = (buf0, buf1, buf2, buf3, buf4, buf5, buf6)
        sems = (sem0, sem1, sem2, sem3, sem4, sem5, sem6)
        for tt, idx, out in ((tt_u, uidx_hbm, ou_hbm),
                             (tt_b, bidx_hbm, ob_hbm)):
            _gather_one_table(tt, idx, out, idxall, cu, cp, glu, glp,
                              fbuf, bufs, sems, ssem,
                              hist_s, cum_s, occb_s, occs_s, wid, lo, hi)

    return sc_gather


_BM = 1024


def _dot(a, b):
    return jax.lax.dot(a, b, preferred_element_type=jnp.float32)


def _mlp_body(ue_ref, be_ref, w1a_ref, w1b_ref, b1_ref, w2_ref, b2_ref,
              w3_ref, b3_ref, w4_ref, b4_ref, out_ref):
    ue = ue_ref[...].astype(jnp.bfloat16)
    be = be_ref[...].astype(jnp.bfloat16)
    h = _dot(ue, w1a_ref[...]) + _dot(be, w1b_ref[...]) + b1_ref[...]
    h = jnp.maximum(h, 0.0).astype(jnp.bfloat16)
    h = jnp.maximum(_dot(h, w2_ref[...]) + b2_ref[...], 0.0).astype(jnp.bfloat16)
    h = jnp.maximum(_dot(h, w3_ref[...]) + b3_ref[...], 0.0)
    out_ref[...] = jnp.sum(h * w4_ref[...], axis=1) + b4_ref[0]


def _mlp(ue, be, W1a, W1b, b1, W2, b2, W3, b3, w4row, b4):
    grid = (BATCH // _BM,)
    full = lambda i: (0, 0)
    return pl.pallas_call(
        _mlp_body,
        grid=grid,
        in_specs=[
            pl.BlockSpec((_BM, D), lambda i: (i, 0)),
            pl.BlockSpec((_BM, D), lambda i: (i, 0)),
            pl.BlockSpec((D, 1024), full),
            pl.BlockSpec((D, 1024), full),
            pl.BlockSpec((1, 1024), full),
            pl.BlockSpec((1024, 512), full),
            pl.BlockSpec((1, 512), full),
            pl.BlockSpec((512, 256), full),
            pl.BlockSpec((1, 256), full),
            pl.BlockSpec((1, 256), full),
            pl.BlockSpec(memory_space=pltpu.SMEM),
        ],
        out_specs=pl.BlockSpec((_BM,), lambda i: (i,)),
        out_shape=jax.ShapeDtypeStruct((BATCH,), jnp.float32),
    )(ue, be, W1a, W1b, b1, W2, b2, W3, b3, w4row, b4)


def kernel(users, businesses, user_table, business_table,
           W1, b1, W2, b2, W3, b3, W4, b4):
    uidx = users.astype(jnp.int32)
    bidx = businesses.astype(jnp.int32)
    ou, ob = _make_sc_gather()(user_table.T, business_table.T, uidx, bidx)
    ue = ou.reshape(BATCH + 128, D)[:BATCH]
    be = ob.reshape(BATCH + 128, D)[:BATCH]
    W1a = W1[:D].astype(jnp.bfloat16)
    W1b = W1[D:].astype(jnp.bfloat16)
    w4row = W4.reshape(1, 256)
    return _mlp(ue, be, W1a, W1b, b1.reshape(1, 1024),
                W2.astype(jnp.bfloat16), b2.reshape(1, 512),
                W3.astype(jnp.bfloat16), b3.reshape(1, 256), w4row, b4)


# speedup vs baseline: 1.1592x; 1.0064x over previous
import functools

import jax
import jax.numpy as jnp
from jax import lax
from jax.experimental import pallas as pl
from jax.experimental.pallas import tpu as pltpu
from jax.experimental.pallas import tpu_sc as plsc

BATCH = 16384
D = 64
NV = BATCH // 16

_NC, _NS = 2, 16
_NW = _NC * _NS
NBLK = 7813
RPW = 245
NSLOT = 768
TRASH = BATCH
OUT1D = (BATCH + 128) * D
NRING = 7


def _s0(v):
    return lax.squeeze(lax.slice(v, (0,), (1,)), (0,))


def _iota16():
    return lax.broadcasted_iota(jnp.int32, (16,), 0)


def _gather_one_table(tt_hbm, idx_hbm, out_hbm, idxall, cu, cp, glu, glp,
                      fbuf, bufs, sems, ssem, hist_s, cum_s,
                      occb_s, occs_s, wid, lo, hi):
    iota = _iota16()

    pltpu.sync_copy(idx_hbm, idxall)

    def round_body(carry):
        vres, r = carry

        def p1_cond(c):
            v, off = c
            return (v < NV) & (off < NSLOT - 15)

        def p1_body(c):
            v, off = c
            u = idxall[pl.ds(v * 16, 16)]
            p = iota + v * 16
            b = u >> 7
            m = (b >= lo) & (b < hi)
            plsc.store_compressed(cu.at[pl.ds(off, 16)], u, mask=m)
            plsc.store_compressed(cp.at[pl.ds(off, 16)], p, mask=m)
            cnt = _s0(plsc.all_reduce_population_count(m))
            return v + 1, off + cnt

        vres2, n_t = lax.while_loop(p1_cond, p1_body, (vres, 0))

        def p2_zero(i, _):
            hist_s[i] = 0
            return 0
        lax.fori_loop(0, RPW, p2_zero, 0)

        def p2_body(j, _):
            u = _s0(cu[pl.ds(j, 16)])
            bb = (u >> 7) - lo
            hist_s[bb] = hist_s[bb] + 1
            return 0
        lax.fori_loop(0, n_t, p2_body, 0)

        occb_s[0] = lo
        occs_s[0] = 0

        def p3_body(i, c):
            run, no = c
            h = hist_s[i]
            cum_s[i] = run

            @pl.when(h > 0)
            def _():
                occb_s[no] = i + lo
                occs_s[no] = run

            return run + h, jnp.where(h > 0, no + 1, no)

        _, nocc = lax.fori_loop(0, hi - lo, p3_body, (0, 0))

        for w in range(NSLOT // 16):
            glp[pl.ds(16 * w, 16)] = jnp.full((16,), TRASH, jnp.int32) + (
                iota + 16 * w) % 128

        def p4_body(j, c):
            u = _s0(cu[pl.ds(j, 16)])
            p = _s0(cp[pl.ds(j, 16)])
            bb = (u >> 7) - lo
            slot = cum_s[bb]
            cum_s[bb] = slot + 1
            m0 = iota == 0
            sv = jnp.full((16,), 0, jnp.int32) + slot
            plsc.store_scatter(glu, [sv], jnp.full((16,), 0, jnp.int32) + u,
                               mask=m0)
            plsc.store_scatter(glp, [sv], jnp.full((16,), 0, jnp.int32) + p,
                               mask=m0)
            return c
        lax.fori_loop(0, n_t, p4_body, 0)

        def fetch(k, buf, sem):
            kk = jnp.maximum(jnp.minimum(k, nocc - 1), 0)
            off = pl.multiple_of(occb_s[kk] * 128, 128)
            pltpu.async_copy(tt_hbm.at[pl.ds(0, 32), pl.ds(off, 128)],
                             buf.at[pl.ds(0, 32)], sem)
            pltpu.async_copy(tt_hbm.at[pl.ds(32, 32), pl.ds(off, 128)],
                             buf.at[pl.ds(32, 32)], sem)

        for b in range(NRING):
            fetch(b, bufs[b], sems[b])

        def chunk_body(cidx, c):
            for b in range(NRING):
                k = cidx * NRING + b
                for hh in range(2):
                    pltpu.make_async_copy(
                        tt_hbm.at[pl.ds(32 * hh, 32), pl.ds(0, 128)],
                        bufs[b].at[pl.ds(32 * hh, 32)], sems[b]).wait()

                @pl.when(k < nocc)
                def _(k=k, b=b):
                    kk = jnp.maximum(jnp.minimum(k, nocc - 1), 0)
                    st = occs_s[kk]
                    cnt = hist_s[occb_s[kk] - lo]

                    def ex_body(j, cc2):
                        u = _s0(glu[pl.ds(j, 16)])
                        ln = u & 127
                        lnv = jnp.full((16,), 0, jnp.int32) + ln
                        for cc in range(4):
                            vals = plsc.load_gather(
                                bufs[b], [iota + cc * 16, lnv])
                            fbuf[pl.ds(j * D + cc * 16, 16)] = vals
                        p = _s0(glp[pl.ds(j, 16)])
                        src = pl.multiple_of(j * D, D)
                        dst = pl.multiple_of(p * D, D)
                        pltpu.async_copy(fbuf.at[pl.ds(src, D)],
                                         out_hbm.at[pl.ds(dst, D)], ssem)
                        return cc2
                    lax.fori_loop(st, st + cnt, ex_body, 0)

                fetch(k + NRING, bufs[b], sems[b])
            return c
        lax.fori_loop(0, (nocc + NRING - 1) // NRING, chunk_body, 0)
        for b in range(NRING):
            for hh in range(2):
                pltpu.make_async_copy(
                    tt_hbm.at[pl.ds(32 * hh, 32), pl.ds(0, 128)],
                    bufs[b].at[pl.ds(32 * hh, 32)], sems[b]).wait()

        def p7_drain(j, carry):
            pltpu.make_async_copy(
                fbuf.at[pl.ds(0, D)], out_hbm.at[pl.ds(0, D)], ssem).wait()
            return carry
        lax.fori_loop(0, n_t, p7_drain, 0)

        return vres2, r + 1

    def round_cond(carry):
        vres, r = carry
        return vres < NV

    lax.while_loop(round_cond, round_body, (0, 0))


@functools.cache
def _make_sc_gather():
    mesh = plsc.VectorSubcoreMesh(core_axis_name="c", subcore_axis_name="s")

    @functools.partial(
        pl.kernel,
        out_type=(
            jax.ShapeDtypeStruct((OUT1D,), jnp.float32),
            jax.ShapeDtypeStruct((OUT1D,), jnp.float32),
        ),
        mesh=mesh,
        compiler_params=pltpu.CompilerParams(needs_layout_passes=False),
        scratch_types=[
            pltpu.VMEM((BATCH,), jnp.int32),
            pltpu.VMEM((NSLOT + 16,), jnp.int32),
            pltpu.VMEM((NSLOT + 16,), jnp.int32),
            pltpu.VMEM((NSLOT + 16,), jnp.int32),
            pltpu.VMEM((NSLOT + 16,), jnp.int32),
            pltpu.VMEM((NSLOT * D,), jnp.float32),
            pltpu.VMEM((D, 128), jnp.float32),
            pltpu.VMEM((D, 128), jnp.float32),
            pltpu.VMEM((D, 128), jnp.float32),
            pltpu.VMEM((D, 128), jnp.float32),
            pltpu.VMEM((D, 128), jnp.float32),
            pltpu.VMEM((D, 128), jnp.float32),
            pltpu.VMEM((D, 128), jnp.float32),
            pltpu.SemaphoreType.DMA,
            pltpu.SemaphoreType.DMA,
            pltpu.SemaphoreType.DMA,
            pltpu.SemaphoreType.DMA,
            pltpu.SemaphoreType.DMA,
            pltpu.SemaphoreType.DMA,
            pltpu.SemaphoreType.DMA,
            pltpu.SemaphoreType.DMA,

            pltpu.SMEM((256,), jnp.int32),
            pltpu.SMEM((256,), jnp.int32),
            pltpu.SMEM((256,), jnp.int32),
            pltpu.SMEM((256,), jnp.int32),
        ],
    )
    def sc_gather(tt_u, tt_b, uidx_hbm, bidx_hbm, ou_hbm, ob_hbm,
                  idxall, cu, cp, glu, glp, fbuf,
                  buf0, buf1, buf2, buf3, buf4, buf5, buf6,
                  sem0, sem1, sem2, sem3, sem4, sem5, sem6, ssem,
                  hist_s, cum_s, occb_s, occs_s):
        wid = lax.axis_index("s") * _NC + lax.axis_index("c")
        lo = wid * RPW
        hi = jnp.minimum(lo + RPW, NBLK)
        bufs = (buf0, buf1, buf2, buf3, buf4, buf5, buf6)
        sems = (sem0, sem1, sem2, sem3, sem4, sem5, sem6)
        for tt, idx, out in ((tt_u, uidx_hbm, ou_hbm),
                             (tt_b, bidx_hbm, ob_hbm)):
            _gather_one_table(tt, idx, out, idxall, cu, cp, glu, glp,
                              fbuf, bufs, sems, ssem,
                              hist_s, cum_s, occb_s, occs_s, wid, lo, hi)

    return sc_gather


_BM = 1024


def _dot(a, b):
    return jax.lax.dot(a, b, preferred_element_type=jnp.float32)


def _mlp_body(ue_ref, be_ref, w1a_ref, w1b_ref, b1_ref, w2_ref, b2_ref,
              w3_ref, b3_ref, w4_ref, b4_ref, out_ref):
    ue = ue_ref[...].astype(jnp.bfloat16)
    be = be_ref[...].astype(jnp.bfloat16)
    h = _dot(ue, w1a_ref[...]) + _dot(be, w1b_ref[...]) + b1_ref[...]
    h = jnp.maximum(h, 0.0).astype(jnp.bfloat16)
    h = jnp.maximum(_dot(h, w2_ref[...]) + b2_ref[...], 0.0).astype(jnp.bfloat16)
    h = jnp.maximum(_dot(h, w3_ref[...]) + b3_ref[...], 0.0)
    out_ref[...] = jnp.sum(h * w4_ref[...], axis=1) + b4_ref[0]


def _mlp(ue, be, W1a, W1b, b1, W2, b2, W3, b3, w4row, b4):
    grid = (BATCH // _BM,)
    full = lambda i: (0, 0)
    return pl.pallas_call(
        _mlp_body,
        grid=grid,
        in_specs=[
            pl.BlockSpec((_BM, D), lambda i: (i, 0)),
            pl.BlockSpec((_BM, D), lambda i: (i, 0)),
            pl.BlockSpec((D, 1024), full),
            pl.BlockSpec((D, 1024), full),
            pl.BlockSpec((1, 1024), full),
            pl.BlockSpec((1024, 512), full),
            pl.BlockSpec((1, 512), full),
            pl.BlockSpec((512, 256), full),
            pl.BlockSpec((1, 256), full),
            pl.BlockSpec((1, 256), full),
            pl.BlockSpec(memory_space=pltpu.SMEM),
        ],
        out_specs=pl.BlockSpec((_BM,), lambda i: (i,)),
        out_shape=jax.ShapeDtypeStruct((BATCH,), jnp.float32),
    )(ue, be, W1a, W1b, b1, W2, b2, W3, b3, w4row, b4)


def kernel(users, businesses, user_table, business_table,
           W1, b1, W2, b2, W3, b3, W4, b4):
    uidx = users.astype(jnp.int32)
    bidx = businesses.astype(jnp.int32)
    ou, ob = _make_sc_gather()(user_table.T, business_table.T, uidx, bidx)
    ue = ou.reshape(BATCH + 128, D)[:BATCH]
    be = ob.reshape(BATCH + 128, D)[:BATCH]
    W1a = W1[:D].astype(jnp.bfloat16)
    W1b = W1[D:].astype(jnp.bfloat16)
    w4row = W4.reshape(1, 256)
    return _mlp(ue, be, W1a, W1b, b1.reshape(1, 1024),
                W2.astype(jnp.bfloat16), b2.reshape(1, 512),
                W3.astype(jnp.bfloat16), b3.reshape(1, 256), w4row, b4)
